# baseline (speedup 1.0000x reference)
import functools

import jax
import jax.numpy as jnp
from jax import lax
from jax.experimental import pallas as pl
from jax.experimental.pallas import tpu as pltpu
from jax.experimental.pallas import tpu_sc as plsc

_D = 32
_NW = 32
_CH = 128
_NBUF = 4
_GRP = _CH // 16
_OB = _CH * _D


def _sc_gather(x4, xo, table4):
    n = x4.size
    per_w = n // _NW
    steps = per_w // _CH
    groups = steps // _NBUF
    mesh = plsc.VectorSubcoreMesh(core_axis_name="c", subcore_axis_name="s")

    @functools.partial(
        pl.kernel,
        mesh=mesh,
        out_type=jax.ShapeDtypeStruct((n * _D,), jnp.float32),
        scratch_types=[
            pltpu.VMEM((per_w,), jnp.int32),
            pltpu.VMEM((per_w,), jnp.int32),
            pltpu.VMEM((_NBUF * _CH, 128), jnp.float32),
            pltpu.VMEM((_NBUF * _OB,), jnp.float32),
            pltpu.SemaphoreType.DMA((_NBUF,)),
            pltpu.SemaphoreType.DMA((_NBUF,)),
        ],
        compiler_params=pltpu.CompilerParams(needs_layout_passes=False),
    )
    def k(x4_hbm, xo_hbm, tab_hbm, out_hbm, idx_v, xo_v, rows_v, obuf_v, gsem, osem):
        wid = lax.axis_index("s") * 2 + lax.axis_index("c")
        base = wid * per_w
        pltpu.sync_copy(x4_hbm.at[pl.ds(base, per_w)], idx_v)
        pltpu.sync_copy(xo_hbm.at[pl.ds(base, per_w)], xo_v)
        lanes = lax.iota(jnp.int32, 16)

        def extract(j, b):
            def grp(g, carry):
                xov = xo_v[pl.ds(j * _CH + g * 16, 16)]
                rowv = b * _CH + g * 16 + lanes
                obase = b * _OB + (g * 16 + lanes) * _D
                for c in range(_D):
                    v = plsc.load_gather(rows_v, [rowv, xov + c])
                    plsc.store_scatter(obuf_v, [obase + c], v)
                return carry

            lax.fori_loop(0, _GRP, grp, 0)

        def group(g, carry):
            j0 = g * _NBUF
            for b in range(_NBUF):
                pltpu.async_copy(
                    tab_hbm.at[idx_v.at[pl.ds((j0 + b) * _CH, _CH)]],
                    rows_v.at[pl.ds(b * _CH, _CH)],
                    gsem.at[b],
                )
            for b in range(_NBUF):
                j = j0 + b
                pltpu.make_async_copy(
                    tab_hbm.at[idx_v.at[pl.ds(j * _CH, _CH)]],
                    rows_v.at[pl.ds(b * _CH, _CH)],
                    gsem.at[b],
                ).wait()
                extract(j, b)
                pltpu.async_copy(
                    obuf_v.at[pl.ds(b * _OB, _OB)],
                    out_hbm.at[pl.ds((base + j * _CH) * _D, _OB)],
                    osem.at[b],
                )
            for b in range(_NBUF):
                pltpu.make_async_copy(
                    obuf_v.at[pl.ds(b * _OB, _OB)],
                    out_hbm.at[pl.ds((base + (j0 + b) * _CH) * _D, _OB)],
                    osem.at[b],
                ).wait()
            return carry

        lax.fori_loop(0, groups, group, 0)

    return k(x4, xo, table4)


def kernel(x, table):
    b, f = x.shape
    x_flat = x.reshape(-1)
    x4 = x_flat >> 2
    xo = (x_flat & 3) << 5
    table4 = table.reshape(-1, 128)
    out = _sc_gather(x4, xo, table4)
    return out.reshape(b, f, _D)


# speedup vs baseline: 1.6942x; 1.6942x over previous
import functools

import jax
import jax.numpy as jnp
from jax import lax
from jax.experimental import pallas as pl
from jax.experimental.pallas import tpu as pltpu
from jax.experimental.pallas import tpu_sc as plsc

_D = 32
_NW = 32
_BB = 4
_NBUF = 8


def _sc_gather(x_flat, table, bsz, fields):
    n = x_flat.shape[0]
    ch = _BB * fields
    per_w = n // _NW
    steps = per_w // ch
    groups = steps // _NBUF
    b_per_w = bsz // _NW
    mesh = plsc.VectorSubcoreMesh(core_axis_name="c", subcore_axis_name="s")

    @functools.partial(
        pl.kernel,
        mesh=mesh,
        out_type=jax.ShapeDtypeStruct((bsz, fields, _D), jnp.float32),
        scratch_types=[
            pltpu.VMEM((per_w,), jnp.int32),
            pltpu.VMEM((_NBUF, ch, _D), jnp.float32),
            pltpu.SemaphoreType.DMA((_NBUF,)),
            pltpu.SemaphoreType.DMA((_NBUF,)),
        ],
        compiler_params=pltpu.CompilerParams(use_tc_tiling_on_sc=False),
    )
    def k(x_hbm, tab_hbm, out_hbm, idx_v, rows_v, gsem, osem):
        wid = lax.axis_index("s") * 2 + lax.axis_index("c")
        base = wid * per_w
        bbase = wid * b_per_w
        pltpu.sync_copy(x_hbm.at[pl.ds(base, per_w)], idx_v)

        def group(g, carry):
            j0 = g * _NBUF
            for b in range(_NBUF):
                pltpu.async_copy(
                    tab_hbm.at[idx_v.at[pl.ds((j0 + b) * ch, ch)]],
                    rows_v.at[b],
                    gsem.at[b],
                )
            for b in range(_NBUF):
                j = j0 + b
                pltpu.make_async_copy(
                    tab_hbm.at[idx_v.at[pl.ds(j * ch, ch)]],
                    rows_v.at[b],
                    gsem.at[b],
                ).wait()
                for r in range(_BB):
                    pltpu.async_copy(
                        rows_v.at[b].at[pl.ds(r * fields, fields)],
                        out_hbm.at[bbase + j * _BB + r],
                        osem.at[b],
                    )
            for b in range(_NBUF):
                j = j0 + b
                for r in range(_BB):
                    pltpu.make_async_copy(
                        rows_v.at[b].at[pl.ds(r * fields, fields)],
                        out_hbm.at[bbase + j * _BB + r],
                        osem.at[b],
                    ).wait()
            return carry

        lax.fori_loop(0, groups, group, 0)

    return k(x_flat, table)


def kernel(x, table):
    b, f = x.shape
    return _sc_gather(x.reshape(-1), table, b, f)


# speedup vs baseline: 2.1354x; 1.2604x over previous
import functools

import jax
import jax.numpy as jnp
from jax import lax
from jax.experimental import pallas as pl
from jax.experimental.pallas import tpu as pltpu
from jax.experimental.pallas import tpu_sc as plsc

_D = 32
_NW = 32
_BB = 4
_NBUF = 8


def _sc_gather(x_flat, table, bsz, fields):
    n = x_flat.shape[0]
    ch = _BB * fields
    per_w = n // _NW
    steps = per_w // ch
    groups = steps // _NBUF
    b_per_w = bsz // _NW
    mesh = plsc.VectorSubcoreMesh(core_axis_name="c", subcore_axis_name="s")

    @functools.partial(
        pl.kernel,
        mesh=mesh,
        out_type=jax.ShapeDtypeStruct((bsz, 32, 128), jnp.float32),
        scratch_types=[
            pltpu.VMEM((per_w,), jnp.int32),
            pltpu.VMEM((_NBUF, ch, _D), jnp.float32),
            pltpu.SemaphoreType.DMA((_NBUF,)),
            pltpu.SemaphoreType.DMA((_NBUF,)),
        ],
        compiler_params=pltpu.CompilerParams(use_tc_tiling_on_sc=False),
    )
    def k(x_hbm, tab_hbm, out_hbm, idx_v, rows_v, gsem, osem):
        wid = lax.axis_index("s") * 2 + lax.axis_index("c")
        base = wid * per_w
        bbase = wid * b_per_w
        pltpu.sync_copy(x_hbm.at[pl.ds(base, per_w)], idx_v)

        def group(g, carry):
            j0 = g * _NBUF
            for b in range(_NBUF):
                pltpu.async_copy(
                    tab_hbm.at[idx_v.at[pl.ds((j0 + b) * ch, ch)]],
                    rows_v.at[b],
                    gsem.at[b],
                )
            for b in range(_NBUF):
                j = j0 + b
                pltpu.make_async_copy(
                    tab_hbm.at[idx_v.at[pl.ds(j * ch, ch)]],
                    rows_v.at[b],
                    gsem.at[b],
                ).wait()
                for r in range(_BB):
                    pltpu.async_copy(
                        rows_v.at[b].at[pl.ds(r * fields, fields)],
                        out_hbm.at[bbase + j * _BB + r, pl.ds(0, fields), pl.ds(0, _D)],
                        osem.at[b],
                    )
            for b in range(_NBUF):
                j = j0 + b
                for r in range(_BB):
                    pltpu.make_async_copy(
                        rows_v.at[b].at[pl.ds(r * fields, fields)],
                        out_hbm.at[bbase + j * _BB + r, pl.ds(0, fields), pl.ds(0, _D)],
                        osem.at[b],
                    ).wait()
            return carry

        lax.fori_loop(0, groups, group, 0)

    return k(x_flat, table)


def kernel(x, table):
    b, f = x.shape
    out = _sc_gather(x.reshape(-1), table, b, f)
    return out[:, :f, :_D]
